# baseline (speedup 1.0000x reference)
import functools

import numpy as np
import jax
import jax.numpy as jnp
from jax import lax
from jax.experimental import pallas as pl
from jax.experimental.pallas import tpu as pltpu
from jax.experimental.pallas import tpu_sc as plsc

MAXD = 7
H = 128
ND = [4 ** d for d in range(MAXD + 1)]
NWORK = 32
TAPS = [(dy, dx) for dy in (-1, 0, 1) for dx in (-1, 0, 1)]
BF16 = jnp.bfloat16


def _interleave(x):
    x = np.asarray(x, np.int64) & 0xFFFF
    x = (x | (x << 8)) & 0x00FF00FF
    x = (x | (x << 4)) & 0x0F0F0F0F
    x = (x | (x << 2)) & 0x33333333
    x = (x | (x << 1)) & 0x55555555
    return x


def _tables(d):
    n = 1 << d
    N = n * n
    i = np.arange(N)
    x, y = i % n, i // n
    morton_of_grid = (_interleave(x) | (_interleave(y) << 1)).astype(np.int32)
    grid_of_morton = np.empty(N, np.int32)
    grid_of_morton[morton_of_grid] = i.astype(np.int32)
    xc = (x.astype(np.float32) + 0.5) / np.float32(n)
    yc = (y.astype(np.float32) + 0.5) / np.float32(n)
    dn = np.full(N, np.float32(d) / np.float32(MAXD), np.float32)
    pos = np.stack([xc, yc, dn], 1)
    freqs = (2.0 ** np.arange(6)).astype(np.float32).reshape(1, 1, -1)
    xx = pos[..., None].astype(np.float32) * np.float32(np.pi) * 2.0 * freqs
    enc = np.concatenate([np.sin(xx), np.cos(xx)], -1).reshape(N, -1)
    posf = np.concatenate([pos, enc], 1).astype(np.float32)
    masks = np.stack([(x > 0).astype(np.float32),
                      (x < n - 1).astype(np.float32)], 1)
    return morton_of_grid, grid_of_morton, np.concatenate([posf, masks], 1)


_TABS = [_tables(d) for d in range(MAXD + 1)]
_POS7_M = _TABS[7][2][:, :39][_TABS[7][1]]

_P_ONEHOT = {}
for _d in range(2, 7):
    _N = ND[_d]
    _p = np.zeros((_N, _N), np.float32)
    _p[np.arange(_N), _TABS[_d][0]] = 1.0
    _P_ONEHOT[_d] = _p

_SX = {}
for _d in range(1, MAXD):
    _n = 1 << _d
    _m = _n // 2
    _s = np.zeros((max(_m, 8), _n), np.float32)
    for _X in range(_m):
        _s[_X, 2 * _X] = 0.25
        _s[_X, 2 * _X + 1] = 0.25
    _SX[_d] = _s

_S4 = np.zeros((128, 512), np.float32)
for _r in range(128):
    _S4[_r, 4 * _r:4 * _r + 4] = 0.25


@functools.lru_cache(maxsize=None)
def _sc_mesh():
    return plsc.VectorSubcoreMesh(core_axis_name="c", subcore_axis_name="s",
                                  num_cores=2, num_subcores=16)


_E_DEPTHS = [6, 5, 4, 3, 2]
_E_ALL = {6: 128, 5: 32, 4: 8}
_E_W0 = {3: 64, 2: 16}
_E_BOFF = {6: 0, 5: 128, 4: 160, 3: 168, 2: 232}
_E_BUF = 248


def _sc_unshuffle_body(*refs):
    ne = len(_E_DEPTHS)
    e_refs = dict(zip(_E_DEPTHS, refs[0:ne]))
    i_refs = dict(zip(_E_DEPTHS, refs[ne:2 * ne]))
    o_refs = dict(zip(_E_DEPTHS, refs[2 * ne:3 * ne]))
    idx_v, rows_v, sem = refs[3 * ne:]
    w = lax.axis_index("s") * 2 + lax.axis_index("c")

    for d, rw in _E_ALL.items():
        boff = _E_BOFF[d]
        pltpu.sync_copy(i_refs[d].at[pl.ds(w * rw, rw)],
                        idx_v.at[pl.ds(boff, rw)])
    descs = []
    for d, rw in _E_ALL.items():
        boff = _E_BOFF[d]
        descs.append(pltpu.async_copy(
            e_refs[d].at[idx_v.at[pl.ds(boff, rw)]],
            rows_v.at[pl.ds(boff, rw)], sem))
    for de in descs:
        de.wait()
    for d, rw in _E_ALL.items():
        boff = _E_BOFF[d]
        pltpu.sync_copy(rows_v.at[pl.ds(boff, rw)],
                        o_refs[d].at[pl.ds(w * rw, rw)])

    @pl.when(w == 0)
    def _():
        for d, rw in _E_W0.items():
            boff = _E_BOFF[d]
            pltpu.sync_copy(i_refs[d], idx_v.at[pl.ds(boff, rw)])
        descs0 = []
        for d, rw in _E_W0.items():
            boff = _E_BOFF[d]
            descs0.append(pltpu.async_copy(
                e_refs[d].at[idx_v.at[pl.ds(boff, rw)]],
                rows_v.at[pl.ds(boff, rw)], sem))
        for de in descs0:
            de.wait()
        for d, rw in _E_W0.items():
            boff = _E_BOFF[d]
            pltpu.sync_copy(rows_v.at[pl.ds(boff, rw)], o_refs[d])


@functools.lru_cache(maxsize=None)
def _sc_unshuffle():
    return pl.kernel(
        _sc_unshuffle_body,
        out_type=tuple(jax.ShapeDtypeStruct((ND[d], H), jnp.float32)
                       for d in _E_DEPTHS),
        mesh=_sc_mesh(),
        scratch_types=[
            pltpu.VMEM((_E_BUF,), jnp.int32),
            pltpu.VMEM((_E_BUF, H), jnp.float32),
            pltpu.SemaphoreType.DMA,
        ],
    )


def _shift_zero(T, s, N):
    if s == 0:
        return T
    z = jnp.zeros((abs(s), T.shape[1]), T.dtype)
    if s > 0:
        return jnp.concatenate([T[s:], z], axis=0)
    return jnp.concatenate([z, T[:s]], axis=0)


def _layernorm(z, g2, b2):
    mu = jnp.mean(z, axis=1, keepdims=True)
    zc = z - mu
    var = jnp.mean(zc * zc, axis=1, keepdims=True)
    return zc * lax.rsqrt(var + 1e-5) * g2 + b2


def _tca_body(X7, inW, inb, embW7, embb7, g27, b27, S4, f6, W0row, E7, P6):
    h = jnp.dot(X7[...].astype(BF16), inW[...],
                preferred_element_type=jnp.float32) + inb[...]
    E7[...] = h
    for c in range(32):
        blk = E7[pl.ds(c * 512, 512), :].astype(BF16)
        P6[pl.ds(c * 128, 128), :] = jnp.dot(S4[...], blk,
                                             preferred_element_type=jnp.float32)
    P6[...] = P6[...] + f6[...] * W0row[...]
    z = jnp.dot(h.astype(BF16), embW7[...],
                preferred_element_type=jnp.float32) + embb7[...]
    E7[...] = _layernorm(z, g27[...], b27[...])


_tca = pl.pallas_call(
    _tca_body,
    out_shape=(jax.ShapeDtypeStruct((ND[7], H), jnp.float32),
               jax.ShapeDtypeStruct((ND[6], H), jnp.float32)),
)


def _tcb_body(*refs):
    it = iter(refs)
    X = [next(it) for _ in range(7)]
    P6aug = next(it)
    P6one = next(it)
    Pone = {d: next(it) for d in range(2, 6)}
    fmor = {d: next(it) for d in range(2, 6)}
    W0row = next(it)
    inW = next(it)
    inb = next(it)
    convW = next(it)
    convb = next(it)
    embW = next(it)
    embb = next(it)
    g2 = next(it)
    b2 = next(it)
    Sx = {d: next(it) for d in range(1, 7)}
    E = [next(it) for _ in range(7)]

    Wv = inW[...]
    bv = inb[...]
    for d in range(7):
        A = X[d][...][:, 0:40].astype(BF16)
        h0 = jnp.dot(A, Wv, preferred_element_type=jnp.float32) + bv
        if 2 <= d <= 5:
            ft = (fmor[d][...] * W0row[...]).astype(BF16)
            h0 = h0 + jnp.dot(Pone[d][...], ft,
                              preferred_element_type=jnp.float32)
        if d == 6:
            h0 = h0 + jnp.dot(P6one[...], P6aug[...].astype(BF16),
                              preferred_element_type=jnp.float32)
        E[d][...] = h0

    def conv(dc):
        nc = 1 << dc
        Nc = ND[dc]
        hv = E[dc][...].astype(BF16)
        Xm = X[dc][:, 40:42]
        sums = []
        for dx in (-1, 0, 1):
            t = None
            for dy in (-1, 0, 1):
                j = (dy + 1) * 3 + (dx + 1)
                T = jnp.dot(hv, convW[dc - 1, j * H:(j + 1) * H, :],
                            preferred_element_type=jnp.float32)
                T = _shift_zero(T, dy * nc + dx, Nc)
                t = T if t is None else t + T
            sums.append(t)
        acc = (convb[dc - 1:dc, :] + sums[1]
               + Xm[:, 0:1] * sums[0] + Xm[:, 1:2] * sums[2])
        E[dc][...] = jnp.maximum(acc, 0.0)

    conv(6)
    for d in range(6, 0, -1):
        n = 1 << d
        m = n // 2
        Sxv = Sx[d][...]
        for Y in range(m):
            rA = E[d][pl.ds((2 * Y) * n, n), :]
            rB = E[d][pl.ds((2 * Y + 1) * n, n), :]
            ch = jnp.dot(Sxv, (rA + rB).astype(BF16),
                         preferred_element_type=jnp.float32)
            E[d - 1][pl.ds(Y * m, m), :] = E[d - 1][pl.ds(Y * m, m), :] + ch[:m]
        if d - 1 >= 1:
            conv(d - 1)

    for d in range(7):
        hv = E[d][...].astype(BF16)
        z = jnp.dot(hv, embW[d], preferred_element_type=jnp.float32) + embb[d:d + 1, :]
        E[d][...] = _layernorm(z, g2[d:d + 1, :], b2[d:d + 1, :])


_tcb = pl.pallas_call(
    _tcb_body,
    out_shape=tuple(jax.ShapeDtypeStruct((ND[d], H), jnp.float32)
                    for d in range(7)),
)


def kernel(features_0, features_1, features_2, features_3, features_4,
           features_5, features_6, features_7, in_proj_W, in_proj_b,
           conv_W, conv_b, emb_W, emb_b, ln_g, ln_b, depth_gain):
    feats = [features_0, features_1, features_2, features_3, features_4,
             features_5, features_6, features_7]
    f32 = jnp.float32

    inW16 = in_proj_W.astype(BF16)
    inb2 = in_proj_b.reshape(1, H)
    embW16 = emb_W.astype(BF16)
    g2 = depth_gain[:, None] * ln_g
    b2 = depth_gain[:, None] * ln_b

    X7 = jnp.concatenate([feats[7], jnp.asarray(_POS7_M)], axis=1)
    W0row = in_proj_W[0:1, :]
    E7, P6aug = _tca(X7, inW16, inb2, embW16[7], emb_b[7].reshape(1, H),
                     g2[7].reshape(1, H), b2[7].reshape(1, H),
                     jnp.asarray(_S4).astype(BF16), feats[6], W0row)

    ops = []
    for d in range(7):
        fcol = feats[d] if d < 2 else jnp.zeros((ND[d], 1), f32)
        ops.append(jnp.concatenate([fcol, jnp.asarray(_TABS[d][2])], axis=1))
    ops.append(P6aug)
    ops.append(jnp.asarray(_P_ONEHOT[6]).astype(BF16))
    for d in range(2, 6):
        ops.append(jnp.asarray(_P_ONEHOT[d]).astype(BF16))
    for d in range(2, 6):
        ops.append(feats[d])
    ops.append(W0row)
    ops.append(inW16)
    ops.append(inb2)
    ops.append(conv_W[1:7].astype(BF16))
    ops.append(conv_b[1:7])
    ops.append(embW16[:7])
    ops.append(emb_b[:7])
    ops.append(g2[:7])
    ops.append(b2[:7])
    for d in range(1, 7):
        ops.append(jnp.asarray(_SX[d]).astype(BF16))
    Eg = _tcb(*ops)

    Em = _sc_unshuffle()(
        *[Eg[d] for d in _E_DEPTHS],
        *[jnp.asarray(_TABS[d][1]) for d in _E_DEPTHS])
    EmD = dict(zip(_E_DEPTHS, Em))

    return (Eg[0], Eg[1], EmD[2], EmD[3], EmD[4], EmD[5], EmD[6], E7)
